# baseline (speedup 1.0000x reference)
import functools

import jax
import jax.numpy as jnp
from jax import lax
from jax.experimental import pallas as pl
from jax.experimental.pallas import tpu as pltpu
from jax.experimental.pallas import tpu_sc as plsc

NUM_CORES = 2
NUM_SUBCORES = 16
LANES = 16
NUM_WORKERS = NUM_CORES * NUM_SUBCORES

BATCH = 16384
FACTORS = 64
B_PER_W = BATCH // NUM_WORKERS
GROUPS = B_PER_W // LANES
GLOBAL_MEAN = 3.5


@functools.partial(
    pl.kernel,
    out_type=jax.ShapeDtypeStruct((BATCH,), jnp.float32),
    mesh=plsc.VectorSubcoreMesh(core_axis_name="c", subcore_axis_name="s"),
    compiler_params=pltpu.CompilerParams(needs_layout_passes=False,
                                         use_tc_tiling_on_sc=False),
    scratch_types=[
        pltpu.VMEM((B_PER_W,), jnp.int32),
        pltpu.VMEM((B_PER_W,), jnp.int32),
        pltpu.VMEM((B_PER_W, FACTORS), jnp.float32),
        pltpu.VMEM((B_PER_W, FACTORS), jnp.float32),
        pltpu.VMEM((B_PER_W,), jnp.float32),
        pltpu.VMEM((B_PER_W,), jnp.float32),
        pltpu.VMEM((B_PER_W,), jnp.float32),
        pltpu.SemaphoreType.DMA,
        pltpu.SemaphoreType.DMA,
    ],
)
def _mf_sc_kernel(users_hbm, items_hbm, uemb_hbm, iemb_hbm, ubias_hbm,
                  ibias_hbm, out_hbm, uidx_v, iidx_v, urows_v, irows_v,
                  ub_v, ib_v, out_v, sem_u, sem_i):
    wid = lax.axis_index("s") * NUM_CORES + lax.axis_index("c")
    base = wid * B_PER_W

    pltpu.sync_copy(users_hbm.at[pl.ds(base, B_PER_W)], uidx_v)
    pltpu.sync_copy(items_hbm.at[pl.ds(base, B_PER_W)], iidx_v)

    cu = pltpu.async_copy(uemb_hbm.at[uidx_v], urows_v, sem_u)
    ci = pltpu.async_copy(iemb_hbm.at[iidx_v], irows_v, sem_i)
    cub = pltpu.async_copy(ubias_hbm.at[uidx_v], ub_v, sem_u)
    cib = pltpu.async_copy(ibias_hbm.at[iidx_v], ib_v, sem_i)
    cu.wait()
    ci.wait()
    cub.wait()
    cib.wait()

    lane = lax.iota(jnp.int32, LANES)

    def group_body(g, carry):
        gbase = g * LANES
        acc = (ub_v[pl.ds(gbase, LANES)] + ib_v[pl.ds(gbase, LANES)]
               + GLOBAL_MEAN)
        for rr in range(LANES):
            r = gbase + rr
            partial = jnp.zeros((LANES,), jnp.float32)
            for k in range(FACTORS // LANES):
                u = urows_v[r, pl.ds(k * LANES, LANES)]
                v = irows_v[r, pl.ds(k * LANES, LANES)]
                partial = partial + u * v
            s = jnp.sum(partial)
            acc = jnp.where(lane == rr, acc + s, acc)
        out_v[pl.ds(gbase, LANES)] = acc
        return carry

    lax.fori_loop(0, GROUPS, group_body, 0)

    pltpu.sync_copy(out_v, out_hbm.at[pl.ds(base, B_PER_W)])


def kernel(users, items, user_emb, item_emb, user_bias, item_bias):
    return _mf_sc_kernel(users.astype(jnp.int32), items.astype(jnp.int32),
                         user_emb, item_emb,
                         user_bias.reshape(-1), item_bias.reshape(-1))
